# baseline (speedup 1.0000x reference)
import functools

import jax
import jax.numpy as jnp
from jax import lax
from jax.experimental import pallas as pl
from jax.experimental.pallas import tpu as pltpu
from jax.experimental.pallas import tpu_sc as plsc

_NODES = 1536
_NC, _NS = 2, 16
_NW = _NC * _NS
_ROW = 128


@functools.lru_cache(maxsize=None)
def _build_scatter(E, ACC, CH):
    e_per_tile = E // _NW
    n_chunks = e_per_tile // CH
    acc_per_tile = ACC // _NS
    ZB = 4096
    assert E % _NW == 0 and e_per_tile % CH == 0
    assert ACC % (_NS * ZB) == 0

    n_half = n_chunks // 2
    assert n_half >= 2

    def body(x_hbm, src_hbm, dst_hbm, partials_hbm,
             acc, src_v0, src_v1, dst_v0, dst_v1, vals_v0, vals_v1, zbuf,
             isem, dsem, gsem, ssem, zsem):
        c = lax.axis_index("c")
        s = lax.axis_index("s")
        wid = s * _NC + c
        e0 = wid * e_per_tile

        src_b = (src_v0, src_v1)
        dst_b = (dst_v0, dst_v1)
        val_b = (vals_v0, vals_v1)

        def src_go(j, p):
            pltpu.async_copy(src_hbm.at[pl.ds(e0 + j * CH, CH)],
                             src_b[p], isem.at[p])

        def src_dn(p):
            pltpu.make_async_copy(src_hbm.at[pl.ds(0, CH)],
                                  src_b[p], isem.at[p]).wait()

        def dst_go(j, p):
            pltpu.async_copy(dst_hbm.at[pl.ds(e0 + j * CH, CH)],
                             dst_b[p], dsem.at[p])

        def dst_dn(p):
            pltpu.make_async_copy(dst_hbm.at[pl.ds(0, CH)],
                                  dst_b[p], dsem.at[p]).wait()

        def gat_go(p):
            pltpu.async_copy(x_hbm.at[src_b[p]], val_b[p], gsem.at[p])

        def gat_dn(p):
            pltpu.make_async_copy(x_hbm.at[src_b[p]], val_b[p],
                                  gsem.at[p]).wait()

        def sca_go(p):
            pltpu.async_copy(val_b[p], acc.at[dst_b[p]], ssem.at[p],
                             add=True)

        def sca_dn(p):
            pltpu.make_async_copy(val_b[p], acc.at[dst_b[p]],
                                  ssem.at[p]).wait()

        src_go(0, 0)
        dst_go(0, 0)
        src_go(1, 1)
        dst_go(1, 1)

        def zb16(i, carry):
            zbuf[pl.ds(i * 16, 16)] = jnp.zeros((16,), jnp.float32)
            return carry
        lax.fori_loop(0, ZB // 16, zb16, 0)

        zeros = [pltpu.async_copy(
            zbuf, acc.at[pl.ds(s * acc_per_tile + i * ZB, ZB)], zsem)
            for i in range(acc_per_tile // ZB)]

        src_dn(0)
        gat_go(0)

        gat_dn(0)
        dst_dn(0)
        for z in zeros:
            z.wait()
        plsc.subcore_barrier()
        sca_go(0)
        src_dn(1)
        gat_go(1)
        src_go(2, 0)
        gat_dn(1)
        sca_dn(0)
        dst_go(2, 0)
        src_dn(0)
        gat_go(0)
        dst_dn(1)
        sca_go(1)
        src_go(3, 1)

        def steady(jj, carry):
            a = 2 * jj
            gat_dn(0)
            dst_dn(0)
            sca_go(0)
            src_dn(1)
            sca_dn(1)
            dst_go(a + 1, 1)
            gat_go(1)
            src_go(a + 2, 0)
            gat_dn(1)
            sca_dn(0)
            dst_go(a + 2, 0)
            src_dn(0)
            gat_go(0)
            dst_dn(1)
            sca_go(1)
            src_go(a + 3, 1)
            return carry

        lax.fori_loop(1, n_half - 1, steady, 0)

        a = 2 * (n_half - 1)
        gat_dn(0)
        dst_dn(0)
        sca_go(0)
        src_dn(1)
        sca_dn(1)
        dst_go(a + 1, 1)
        gat_go(1)
        gat_dn(1)
        sca_dn(0)
        dst_dn(1)
        sca_go(1)
        sca_dn(1)

        plsc.subcore_barrier()
        pltpu.sync_copy(acc.at[pl.ds(s * acc_per_tile, acc_per_tile)],
                        partials_hbm.at[c, pl.ds(s * acc_per_tile, acc_per_tile)])

    return pl.kernel(
        body,
        out_type=jax.ShapeDtypeStruct((_NC, ACC), jnp.float32),
        mesh=plsc.VectorSubcoreMesh(core_axis_name="c", subcore_axis_name="s"),
        scratch_types=[
            pltpu.VMEM_SHARED((ACC,), jnp.float32),
            pltpu.VMEM((CH,), jnp.int32),
            pltpu.VMEM((CH,), jnp.int32),
            pltpu.VMEM((CH,), jnp.int32),
            pltpu.VMEM((CH,), jnp.int32),
            pltpu.VMEM((CH,), jnp.float32),
            pltpu.VMEM((CH,), jnp.float32),
            pltpu.VMEM((ZB,), jnp.float32),
            pltpu.SemaphoreType.DMA((2,)),
            pltpu.SemaphoreType.DMA((2,)),
            pltpu.SemaphoreType.DMA((2,)),
            pltpu.SemaphoreType.DMA((2,)),
            pltpu.SemaphoreType.DMA,
        ],
    )


@functools.lru_cache(maxsize=None)
def _build_combine(B, BB):
    def body(w_ref, p_ref, base_ref, x_ref):
        w = w_ref[0, 0]
        x_ref[...] = (p_ref[0] + p_ref[1]) * w + base_ref[...]

    return pl.pallas_call(
        body,
        grid=(B // BB,),
        in_specs=[
            pl.BlockSpec(memory_space=pltpu.SMEM),
            pl.BlockSpec((_NC, BB, _NODES), lambda i: (0, i, 0)),
            pl.BlockSpec((BB, _NODES), lambda i: (i, 0)),
        ],
        out_specs=pl.BlockSpec((BB, _NODES), lambda i: (i, 0)),
        out_shape=jax.ShapeDtypeStruct((B, _NODES), jnp.float32),
    )


def kernel(initial_llrs, edge_index, weights):
    B, nv = initial_llrs.shape
    ACC = B * _NODES
    E = edge_index.shape[1]
    base = jnp.concatenate(
        [initial_llrs, jnp.zeros((B, _NODES - nv), initial_llrs.dtype)], axis=1)
    src = edge_index[0]
    dst = edge_index[1]
    scatter = _build_scatter(E, ACC, 4096)
    combine = _build_combine(B, 256)
    x = base
    outs = []
    for i in range(weights.shape[0]):
        partials = scatter(x.reshape(-1), src, dst)
        x = combine(weights[i].reshape(1, 1),
                    partials.reshape(_NC, B, _NODES), base)
        outs.append(x[:, :nv])
    return tuple(outs)


# speedup vs baseline: 1.0041x; 1.0041x over previous
import functools

import jax
import jax.numpy as jnp
from jax import lax
from jax.experimental import pallas as pl
from jax.experimental.pallas import tpu as pltpu
from jax.experimental.pallas import tpu_sc as plsc

_NODES = 1536
_NC, _NS = 2, 16
_NW = _NC * _NS
_ROW = 128


@functools.lru_cache(maxsize=None)
def _build_scatter(E, ACC, CH):
    e_per_tile = E // _NW
    n_chunks = e_per_tile // CH
    acc_per_tile = ACC // _NS
    ZB = 4096
    assert E % _NW == 0 and e_per_tile % CH == 0
    assert ACC % (_NS * ZB) == 0

    n_half = n_chunks // 2
    assert n_half >= 2

    def body(x_hbm, src_hbm, dst_hbm, partials_hbm,
             acc, src_v0, src_v1, dst_v0, dst_v1, vals_v0, vals_v1, zbuf,
             isem, dsem, gsem, ssem, zsem):
        c = lax.axis_index("c")
        s = lax.axis_index("s")
        wid = s * _NC + c
        e0 = wid * e_per_tile

        src_b = (src_v0, src_v1)
        dst_b = (dst_v0, dst_v1)
        val_b = (vals_v0, vals_v1)

        def src_go(j, p):
            pltpu.async_copy(src_hbm.at[pl.ds(e0 + j * CH, CH)],
                             src_b[p], isem.at[p])

        def src_dn(p):
            pltpu.make_async_copy(src_hbm.at[pl.ds(0, CH)],
                                  src_b[p], isem.at[p]).wait()

        def dst_go(j, p):
            pltpu.async_copy(dst_hbm.at[pl.ds(e0 + j * CH, CH)],
                             dst_b[p], dsem.at[p])

        def dst_dn(p):
            pltpu.make_async_copy(dst_hbm.at[pl.ds(0, CH)],
                                  dst_b[p], dsem.at[p]).wait()

        def gat_go(p):
            pltpu.async_copy(x_hbm.at[src_b[p]], val_b[p], gsem.at[p])

        def gat_dn(p):
            pltpu.make_async_copy(x_hbm.at[src_b[p]], val_b[p],
                                  gsem.at[p]).wait()

        def sca_go(p):
            pltpu.async_copy(val_b[p], acc.at[dst_b[p]], ssem.at[p],
                             add=True)

        def sca_dn(p):
            pltpu.make_async_copy(val_b[p], acc.at[dst_b[p]],
                                  ssem.at[p]).wait()

        src_go(0, 0)
        dst_go(0, 0)
        src_go(1, 1)
        dst_go(1, 1)

        def zb16(i, carry):
            zbuf[pl.ds(i * 16, 16)] = jnp.zeros((16,), jnp.float32)
            return carry
        lax.fori_loop(0, ZB // 16, zb16, 0)

        zeros = [pltpu.async_copy(
            zbuf, acc.at[pl.ds(s * acc_per_tile + i * ZB, ZB)], zsem)
            for i in range(acc_per_tile // ZB)]

        src_dn(0)
        gat_go(0)

        gat_dn(0)
        dst_dn(0)
        for z in zeros:
            z.wait()
        plsc.subcore_barrier()
        sca_go(0)
        src_dn(1)
        gat_go(1)
        src_go(2, 0)
        gat_dn(1)
        sca_dn(0)
        dst_go(2, 0)
        src_dn(0)
        gat_go(0)
        dst_dn(1)
        sca_go(1)
        src_go(3, 1)

        def steady(jj, carry):
            a = 2 * jj
            gat_dn(0)
            dst_dn(0)
            sca_go(0)
            src_dn(1)
            sca_dn(1)
            dst_go(a + 1, 1)
            gat_go(1)
            src_go(a + 2, 0)
            gat_dn(1)
            sca_dn(0)
            dst_go(a + 2, 0)
            src_dn(0)
            gat_go(0)
            dst_dn(1)
            sca_go(1)
            src_go(a + 3, 1)
            return carry

        lax.fori_loop(1, n_half - 1, steady, 0)

        a = 2 * (n_half - 1)
        gat_dn(0)
        dst_dn(0)
        sca_go(0)
        src_dn(1)
        sca_dn(1)
        dst_go(a + 1, 1)
        gat_go(1)
        gat_dn(1)
        sca_dn(0)
        dst_dn(1)
        sca_go(1)
        sca_dn(1)

        plsc.subcore_barrier()
        pltpu.sync_copy(acc.at[pl.ds(s * acc_per_tile, acc_per_tile)],
                        partials_hbm.at[c, pl.ds(s * acc_per_tile, acc_per_tile)])

    return pl.kernel(
        body,
        out_type=jax.ShapeDtypeStruct((_NC, ACC), jnp.float32),
        mesh=plsc.VectorSubcoreMesh(core_axis_name="c", subcore_axis_name="s"),
        scratch_types=[
            pltpu.VMEM_SHARED((ACC,), jnp.float32),
            pltpu.VMEM((CH,), jnp.int32),
            pltpu.VMEM((CH,), jnp.int32),
            pltpu.VMEM((CH,), jnp.int32),
            pltpu.VMEM((CH,), jnp.int32),
            pltpu.VMEM((CH,), jnp.float32),
            pltpu.VMEM((CH,), jnp.float32),
            pltpu.VMEM((ZB,), jnp.float32),
            pltpu.SemaphoreType.DMA((2,)),
            pltpu.SemaphoreType.DMA((2,)),
            pltpu.SemaphoreType.DMA((2,)),
            pltpu.SemaphoreType.DMA((2,)),
            pltpu.SemaphoreType.DMA,
        ],
    )


@functools.lru_cache(maxsize=None)
def _build_combine(B, NV, BB):
    def body(w_ref, p_ref, base_ref, x_ref, out_ref):
        w = w_ref[0, 0]
        xv = (p_ref[0] + p_ref[1]) * w + base_ref[...]
        x_ref[...] = xv
        out_ref[...] = xv[:, :NV]

    return pl.pallas_call(
        body,
        grid=(B // BB,),
        in_specs=[
            pl.BlockSpec(memory_space=pltpu.SMEM),
            pl.BlockSpec((_NC, BB, _NODES), lambda i: (0, i, 0)),
            pl.BlockSpec((BB, _NODES), lambda i: (i, 0)),
        ],
        out_specs=[
            pl.BlockSpec((BB, _NODES), lambda i: (i, 0)),
            pl.BlockSpec((BB, NV), lambda i: (i, 0)),
        ],
        out_shape=[
            jax.ShapeDtypeStruct((B, _NODES), jnp.float32),
            jax.ShapeDtypeStruct((B, NV), jnp.float32),
        ],
    )


def kernel(initial_llrs, edge_index, weights):
    B, nv = initial_llrs.shape
    ACC = B * _NODES
    E = edge_index.shape[1]
    base = jnp.concatenate(
        [initial_llrs, jnp.zeros((B, _NODES - nv), initial_llrs.dtype)], axis=1)
    src = edge_index[0]
    dst = edge_index[1]
    scatter = _build_scatter(E, ACC, 4096)
    combine = _build_combine(B, nv, 128)
    x = base
    outs = []
    for i in range(weights.shape[0]):
        partials = scatter(x.reshape(-1), src, dst)
        x, out = combine(weights[i].reshape(1, 1),
                         partials.reshape(_NC, B, _NODES), base)
        outs.append(out)
    return tuple(outs)
